# Initial kernel scaffold; baseline (speedup 1.0000x reference)
import jax
import jax.numpy as jnp
from jax.experimental import pallas as pl


def kernel(x, edge_index, edge_type, rel_emb, res_att, W_ww, W_rel):
    raise NotImplementedError("write your pallas kernel here")


# speedup vs baseline: 1.7162x; 1.7162x over previous
import functools

import jax
import jax.numpy as jnp
from jax import lax
from jax.experimental import pallas as pl
from jax.experimental.pallas import tpu as pltpu
from jax.experimental.pallas import tpu_sc as plsc

_NC = 2
_NS = 16
_NW = _NC * _NS
_L = 16
_C = 80
_KS = 25


def _mm_body(rel_ref, ww_ref, wrel_ref, r2_ref, o2_ref):
    r = rel_ref[...]
    r2_ref[...] = lax.dot(r, ww_ref[...], precision=lax.Precision.HIGHEST)
    o2_ref[...] = lax.dot(r, wrel_ref[...], precision=lax.Precision.HIGHEST)


def _matmuls(rel_emb, W_ww, W_rel):
    R, H = rel_emb.shape
    return pl.pallas_call(
        _mm_body,
        out_shape=[
            jax.ShapeDtypeStruct((R, H), jnp.float32),
            jax.ShapeDtypeStruct((R, H), jnp.float32),
        ],
    )(rel_emb, W_ww, W_rel)


def _smax_stat_body(dpp_ref, m_ref, s_ref, msc, ssc):
    i = pl.program_id(0)
    d = jnp.sum(dpp_ref[0], axis=1)
    bm = jnp.max(d)
    bs = jnp.sum(jnp.exp(d - bm))

    @pl.when(i == 0)
    def _():
        msc[0] = bm
        ssc[0] = bs

    @pl.when(i > 0)
    def _():
        om = msc[0]
        os = ssc[0]
        nm = jnp.maximum(om, bm)
        ssc[0] = os * jnp.exp(om - nm) + bs * jnp.exp(bm - nm)
        msc[0] = nm

    @pl.when(i == pl.num_programs(0) - 1)
    def _():
        m_ref[0] = msc[0]
        s_ref[0] = ssc[0]


def _smax_norm_body(dpp_ref, m_ref, s_ref, w_ref):
    d = jnp.sum(dpp_ref[0], axis=1)
    w_ref[...] = (jnp.exp(d - m_ref[0]) / s_ref[0]).reshape(w_ref.shape)


def _softmax_w(dpp):
    E = dpp.shape[0]
    B = E // _KS
    dpp3 = dpp.reshape(_KS, B, _L)
    m, s = pl.pallas_call(
        _smax_stat_body,
        grid=(_KS,),
        in_specs=[pl.BlockSpec((1, B, _L), lambda i: (i, 0, 0))],
        out_specs=[
            pl.BlockSpec(memory_space=pltpu.SMEM),
            pl.BlockSpec(memory_space=pltpu.SMEM),
        ],
        out_shape=[
            jax.ShapeDtypeStruct((1,), jnp.float32),
            jax.ShapeDtypeStruct((1,), jnp.float32),
        ],
        scratch_shapes=[
            pltpu.SMEM((1,), jnp.float32),
            pltpu.SMEM((1,), jnp.float32),
        ],
    )(dpp3)
    w = pl.pallas_call(
        _smax_norm_body,
        grid=(_KS,),
        in_specs=[
            pl.BlockSpec((1, B, _L), lambda i: (i, 0, 0)),
            pl.BlockSpec(memory_space=pltpu.SMEM),
            pl.BlockSpec(memory_space=pltpu.SMEM),
        ],
        out_specs=pl.BlockSpec((1, B // 128, 128), lambda i: (i, 0, 0)),
        out_shape=jax.ShapeDtypeStruct((_KS, B // 128, 128), jnp.float32),
    )(dpp3, m, s)
    return w.reshape(E)


def _combine_body(p_ref, o_ref):
    o_ref[...] = jnp.maximum(p_ref[0] + p_ref[1], 0.0)


def _combine(partial):
    _, N, H = partial.shape
    return pl.pallas_call(
        _combine_body,
        out_shape=jax.ShapeDtypeStruct((N, H), jnp.float32),
    )(partial)


def _dp_body(x_hbm, r2_hbm, src_hbm, typ_hbm, dst_hbm, dpp_hbm,
             src_v, typ_v, dst_v, hs_v, hr_v, ht_v, pacc_v, sem):
    E = dpp_hbm.shape[0]
    epw = E // _NW
    wid = lax.axis_index("s") * _NC + lax.axis_index("c")
    base = wid * epw

    def chunk(i, carry):
        cb = base + i * _C
        pltpu.sync_copy(src_hbm.at[pl.ds(cb, _C)], src_v)
        pltpu.sync_copy(typ_hbm.at[pl.ds(cb, _C)], typ_v)
        pltpu.sync_copy(dst_hbm.at[pl.ds(cb, _C)], dst_v)
        cs = pltpu.async_copy(x_hbm.at[src_v], hs_v, sem)
        cr = pltpu.async_copy(r2_hbm.at[typ_v], hr_v, sem)
        ct = pltpu.async_copy(x_hbm.at[dst_v], ht_v, sem)
        cs.wait()
        cr.wait()
        ct.wait()

        def group(g, c2):
            for e in range(_L):
                row = g * _L + e
                acc = jnp.zeros((_L,), jnp.float32)
                for k in range(8):
                    s = hs_v[row, pl.ds(k * _L, _L)]
                    r = hr_v[row, pl.ds(k * _L, _L)]
                    t = ht_v[row, pl.ds(k * _L, _L)]
                    acc = acc + (s + r) * t
                pacc_v[row, :] = acc
            return c2

        lax.fori_loop(0, _C // _L, group, 0)
        pltpu.sync_copy(pacc_v, dpp_hbm.at[pl.ds(cb, _C)])
        return carry

    lax.fori_loop(0, epw // _C, chunk, 0)


def _pass_dp(x, r2, src, typ, dst):
    E = src.shape[0]
    H = x.shape[1]
    mesh = plsc.VectorSubcoreMesh(
        core_axis_name="c", subcore_axis_name="s",
        num_cores=_NC, num_subcores=_NS)
    f = functools.partial(
        pl.kernel,
        out_type=jax.ShapeDtypeStruct((E, _L), jnp.float32),
        mesh=mesh,
        scratch_types=[
            pltpu.VMEM((_C,), jnp.int32),
            pltpu.VMEM((_C,), jnp.int32),
            pltpu.VMEM((_C,), jnp.int32),
            pltpu.VMEM((_C, H), jnp.float32),
            pltpu.VMEM((_C, H), jnp.float32),
            pltpu.VMEM((_C, H), jnp.float32),
            pltpu.VMEM((_C, _L), jnp.float32),
            pltpu.SemaphoreType.DMA,
        ],
    )(_dp_body)
    return f(x, r2, src, typ, dst)


def _scatter_body(x_hbm, r2_hbm, src_hbm, typ_hbm, dst_hbm, w_hbm, z_hbm,
                  out_hbm, src_v, typ_v, dst_v, hs_v, hr_v, w_v, upd_v,
                  acc_sh, sem):
    E = w_hbm.shape[0]
    N = acc_sh.shape[0]
    epw = E // _NW
    rpt0 = (N // 8 // _NS) * 8
    last = N - rpt0 * (_NS - 1)
    cid = lax.axis_index("c")
    sid = lax.axis_index("s")
    wid = sid * _NC + cid
    base = wid * epw
    row0 = sid * rpt0

    @pl.when(sid < _NS - 1)
    def _():
        pltpu.sync_copy(z_hbm.at[pl.ds(0, rpt0)],
                        acc_sh.at[pl.ds(row0, rpt0)])

    @pl.when(sid == _NS - 1)
    def _():
        pltpu.sync_copy(z_hbm, acc_sh.at[pl.ds(row0, last)])

    plsc.subcore_barrier()

    def chunk(i, carry):
        cb = base + i * _C
        pltpu.sync_copy(src_hbm.at[pl.ds(cb, _C)], src_v)
        pltpu.sync_copy(typ_hbm.at[pl.ds(cb, _C)], typ_v)
        pltpu.sync_copy(dst_hbm.at[pl.ds(cb, _C)], dst_v)
        pltpu.sync_copy(w_hbm.at[pl.ds(cb, _C)], w_v)
        cs = pltpu.async_copy(x_hbm.at[src_v], hs_v, sem)
        cr = pltpu.async_copy(r2_hbm.at[typ_v], hr_v, sem)
        cs.wait()
        cr.wait()

        def group(g, c2):
            w16 = w_v[pl.ds(g * _L, _L)]
            for e in range(_L):
                row = g * _L + e
                wb = jnp.full((_L,), w16[e], jnp.float32)
                for k in range(8):
                    s = hs_v[row, pl.ds(k * _L, _L)]
                    r = hr_v[row, pl.ds(k * _L, _L)]
                    upd_v[row, pl.ds(k * _L, _L)] = (s + r) * wb
            return c2

        lax.fori_loop(0, _C // _L, group, 0)
        pltpu.sync_copy(upd_v, acc_sh.at[dst_v], add=True)
        return carry

    lax.fori_loop(0, epw // _C, chunk, 0)
    plsc.subcore_barrier()

    @pl.when(sid < _NS - 1)
    def _():
        pltpu.sync_copy(acc_sh.at[pl.ds(row0, rpt0)],
                        out_hbm.at[cid, pl.ds(row0, rpt0)])

    @pl.when(sid == _NS - 1)
    def _():
        pltpu.sync_copy(acc_sh.at[pl.ds(row0, last)],
                        out_hbm.at[cid, pl.ds(row0, last)])


def _pass_scatter(x, r2, src, typ, dst, w):
    E = src.shape[0]
    N, H = x.shape
    z = jnp.zeros((N - (N // 8 // _NS) * 8 * (_NS - 1), H), jnp.float32)
    mesh = plsc.VectorSubcoreMesh(
        core_axis_name="c", subcore_axis_name="s",
        num_cores=_NC, num_subcores=_NS)
    f = functools.partial(
        pl.kernel,
        out_type=jax.ShapeDtypeStruct((_NC, N, H), jnp.float32),
        mesh=mesh,
        scratch_types=[
            pltpu.VMEM((_C,), jnp.int32),
            pltpu.VMEM((_C,), jnp.int32),
            pltpu.VMEM((_C,), jnp.int32),
            pltpu.VMEM((_C, H), jnp.float32),
            pltpu.VMEM((_C, H), jnp.float32),
            pltpu.VMEM((_C,), jnp.float32),
            pltpu.VMEM((_C, H), jnp.float32),
            pltpu.VMEM_SHARED((N, H), jnp.float32),
            pltpu.SemaphoreType.DMA,
        ],
    )(_scatter_body)
    return f(x, r2, src, typ, dst, w, z)


def kernel(x, edge_index, edge_type, rel_emb, res_att, W_ww, W_rel):
    r2, o2 = _matmuls(rel_emb, W_ww, W_rel)
    src = edge_index[0]
    dst = edge_index[1]
    dpp = _pass_dp(x, r2, src, edge_type, dst)
    w = _softmax_w(dpp)
    partial = _pass_scatter(x, r2, src, edge_type, dst, w)
    x_e = _combine(partial)
    return (x_e, o2, res_att)


# speedup vs baseline: 3.4862x; 2.0313x over previous
import functools

import jax
import jax.numpy as jnp
from jax import lax
from jax.experimental import pallas as pl
from jax.experimental.pallas import tpu as pltpu
from jax.experimental.pallas import tpu_sc as plsc

_NC = 2
_NS = 16
_NW = _NC * _NS
_L = 16
_C = 80
_C2 = 40
_KS = 25


def _tile_rows(n, sid):
    rpt0 = (n // 8 // _NS) * 8
    last = n - rpt0 * (_NS - 1)
    return rpt0, last, sid * rpt0


def _stage_rows(src_hbm, dst_sh, sid):
    n = src_hbm.shape[0]
    rpt0, last, row0 = _tile_rows(n, sid)

    @pl.when(sid < _NS - 1)
    def _():
        pltpu.sync_copy(src_hbm.at[pl.ds(row0, rpt0)],
                        dst_sh.at[pl.ds(row0, rpt0)])

    @pl.when(sid == _NS - 1)
    def _():
        pltpu.sync_copy(src_hbm.at[pl.ds(row0, last)],
                        dst_sh.at[pl.ds(row0, last)])


def _mm_body(rel_ref, ww_ref, wrel_ref, r2_ref, o2_ref):
    r = rel_ref[...]
    r2_ref[...] = lax.dot(r, ww_ref[...], precision=lax.Precision.HIGHEST)
    o2_ref[...] = lax.dot(r, wrel_ref[...], precision=lax.Precision.HIGHEST)


def _matmuls(rel_emb, W_ww, W_rel):
    R, H = rel_emb.shape
    return pl.pallas_call(
        _mm_body,
        out_shape=[
            jax.ShapeDtypeStruct((R, H), jnp.float32),
            jax.ShapeDtypeStruct((R, H), jnp.float32),
        ],
    )(rel_emb, W_ww, W_rel)


def _smax_stat_body(dpp_ref, m_ref, s_ref, msc, ssc):
    i = pl.program_id(0)
    d = jnp.sum(dpp_ref[0], axis=1)
    bm = jnp.max(d)
    bs = jnp.sum(jnp.exp(d - bm))

    @pl.when(i == 0)
    def _():
        msc[0] = bm
        ssc[0] = bs

    @pl.when(i > 0)
    def _():
        om = msc[0]
        os = ssc[0]
        nm = jnp.maximum(om, bm)
        ssc[0] = os * jnp.exp(om - nm) + bs * jnp.exp(bm - nm)
        msc[0] = nm

    @pl.when(i == pl.num_programs(0) - 1)
    def _():
        m_ref[0] = msc[0]
        s_ref[0] = ssc[0]


def _smax_norm_body(dpp_ref, m_ref, s_ref, w_ref):
    d = jnp.sum(dpp_ref[0], axis=1)
    w_ref[...] = (jnp.exp(d - m_ref[0]) / s_ref[0]).reshape(w_ref.shape)


def _softmax_w(dpp):
    E = dpp.shape[0]
    B = E // _KS
    dpp3 = dpp.reshape(_KS, B, _L)
    m, s = pl.pallas_call(
        _smax_stat_body,
        grid=(_KS,),
        in_specs=[pl.BlockSpec((1, B, _L), lambda i: (i, 0, 0))],
        out_specs=[
            pl.BlockSpec(memory_space=pltpu.SMEM),
            pl.BlockSpec(memory_space=pltpu.SMEM),
        ],
        out_shape=[
            jax.ShapeDtypeStruct((1,), jnp.float32),
            jax.ShapeDtypeStruct((1,), jnp.float32),
        ],
        scratch_shapes=[
            pltpu.SMEM((1,), jnp.float32),
            pltpu.SMEM((1,), jnp.float32),
        ],
    )(dpp3)
    w = pl.pallas_call(
        _smax_norm_body,
        grid=(_KS,),
        in_specs=[
            pl.BlockSpec((1, B, _L), lambda i: (i, 0, 0)),
            pl.BlockSpec(memory_space=pltpu.SMEM),
            pl.BlockSpec(memory_space=pltpu.SMEM),
        ],
        out_specs=pl.BlockSpec((1, B // 128, 128), lambda i: (i, 0, 0)),
        out_shape=jax.ShapeDtypeStruct((_KS, B // 128, 128), jnp.float32),
    )(dpp3, m, s)
    return w.reshape(E)


def _combine_body(p_ref, o_ref):
    o_ref[...] = jnp.maximum(p_ref[0] + p_ref[1], 0.0)


def _combine(partial):
    _, N, H = partial.shape
    return pl.pallas_call(
        _combine_body,
        out_shape=jax.ShapeDtypeStruct((N, H), jnp.float32),
    )(partial)


def _dp_body(x_hbm, r2_hbm, src_hbm, typ_hbm, dst_hbm, dpp_hbm,
             src_a, typ_a, dst_a,
             hs_v, hr_v, ht_v, pacc_v,
             sem_a, sem_b):
    nchk = src_hbm.shape[1]
    cid = lax.axis_index("c")
    sid = lax.axis_index("s")
    wid = sid * _NC + cid

    pltpu.sync_copy(src_hbm.at[wid], src_a)
    pltpu.sync_copy(typ_hbm.at[wid], typ_a)
    pltpu.sync_copy(dst_hbm.at[wid], dst_a)
    plsc.subcore_barrier()

    def start_gathers(i, off, sem):
        pltpu.async_copy(x_hbm.at[src_a.at[i]], hs_v.at[pl.ds(off, _C)], sem)
        pltpu.async_copy(r2_hbm.at[typ_a.at[i]], hr_v.at[pl.ds(off, _C)], sem)
        pltpu.async_copy(x_hbm.at[dst_a.at[i]], ht_v.at[pl.ds(off, _C)], sem)

    def wait_gathers(i, off, sem):
        pltpu.make_async_copy(
            x_hbm.at[src_a.at[i]], hs_v.at[pl.ds(off, _C)], sem).wait()
        pltpu.make_async_copy(
            r2_hbm.at[typ_a.at[i]], hr_v.at[pl.ds(off, _C)], sem).wait()
        pltpu.make_async_copy(
            x_hbm.at[dst_a.at[i]], ht_v.at[pl.ds(off, _C)], sem).wait()

    start_gathers(0, 0, sem_a)

    def chunk(i, carry):
        par = lax.rem(i, 2)
        off = par * _C

        @pl.when(par == 0)
        def _():
            wait_gathers(i, 0, sem_a)

        @pl.when(par == 1)
        def _():
            wait_gathers(i, _C, sem_b)

        @pl.when(jnp.logical_and(par == 0, i + 1 < nchk))
        def _():
            start_gathers(i + 1, _C, sem_b)

        @pl.when(jnp.logical_and(par == 1, i + 1 < nchk))
        def _():
            start_gathers(i + 1, 0, sem_a)

        def group(g, c2):
            for e in range(_L):
                row = off + g * _L + e
                acc = jnp.zeros((_L,), jnp.float32)
                for k in range(8):
                    s = hs_v[row, pl.ds(k * _L, _L)]
                    r = hr_v[row, pl.ds(k * _L, _L)]
                    t = ht_v[row, pl.ds(k * _L, _L)]
                    acc = acc + (s + r) * t
                pacc_v[row, :] = acc
            return c2

        lax.fori_loop(0, _C // _L, group, 0)
        pltpu.sync_copy(pacc_v.at[pl.ds(off, _C)], dpp_hbm.at[wid, i])
        return carry

    lax.fori_loop(0, nchk, chunk, 0)


def _pass_dp(x, r2, src3, typ3, dst3):
    NWk, nchk, C = src3.shape
    N, H = x.shape
    R = r2.shape[0]
    mesh = plsc.VectorSubcoreMesh(
        core_axis_name="c", subcore_axis_name="s",
        num_cores=_NC, num_subcores=_NS)
    f = functools.partial(
        pl.kernel,
        out_type=jax.ShapeDtypeStruct((NWk, nchk, C, _L), jnp.float32),
        mesh=mesh,
        scratch_types=[
            pltpu.VMEM((nchk, C), jnp.int32),
            pltpu.VMEM((nchk, C), jnp.int32),
            pltpu.VMEM((nchk, C), jnp.int32),
            pltpu.VMEM((2 * C, H), jnp.float32),
            pltpu.VMEM((2 * C, H), jnp.float32),
            pltpu.VMEM((2 * C, H), jnp.float32),
            pltpu.VMEM((2 * C, _L), jnp.float32),
            pltpu.SemaphoreType.DMA,
            pltpu.SemaphoreType.DMA,
        ],
    )(_dp_body)
    return f(x, r2, src3, typ3, dst3)


def _scatter_body(x_hbm, r2_hbm, src_hbm, typ_hbm, dst_hbm, w_hbm, z_hbm,
                  out_hbm,
                  src_r, typ_r, dst_r, w_r, hs_v, hr_v,
                  acc_sh, sem_i0, sem_i1, sem_a, sem_b, sem_ua, sem_ub):
    E = src_hbm.shape[0]
    N = acc_sh.shape[0]
    nchk = E // _NW // _C2
    cid = lax.axis_index("c")
    sid = lax.axis_index("s")
    wid = sid * _NC + cid
    base = wid * (E // _NW)
    rpt0, last, row0 = _tile_rows(N, sid)

    @pl.when(sid < _NS - 1)
    def _():
        pltpu.sync_copy(z_hbm.at[pl.ds(0, rpt0)],
                        acc_sh.at[pl.ds(row0, rpt0)])

    @pl.when(sid == _NS - 1)
    def _():
        pltpu.sync_copy(z_hbm, acc_sh.at[pl.ds(row0, last)])

    plsc.subcore_barrier()

    def idx_copies(j, sem):
        s8 = lax.rem(j, 8)
        cb = base + j * _C2
        return [
            pltpu.make_async_copy(src_hbm.at[pl.ds(cb, _C2)],
                                  src_r.at[s8], sem),
            pltpu.make_async_copy(typ_hbm.at[pl.ds(cb, _C2)],
                                  typ_r.at[s8], sem),
            pltpu.make_async_copy(dst_hbm.at[pl.ds(cb, _C2)],
                                  dst_r.at[s8], sem),
            pltpu.make_async_copy(w_hbm.at[pl.ds(cb, _C2)],
                                  w_r.at[s8, pl.ds(0, _C2)], sem),
        ]

    def gather_copies(j, sem):
        s8 = lax.rem(j, 8)
        hoff = lax.rem(j, 3) * _C2
        roff = lax.rem(j, 2) * _C2
        return [
            pltpu.make_async_copy(x_hbm.at[src_r.at[s8]],
                                  hs_v.at[pl.ds(hoff, _C2)], sem),
            pltpu.make_async_copy(r2_hbm.at[typ_r.at[s8]],
                                  hr_v.at[pl.ds(roff, _C2)], sem),
        ]

    def scatter_start(j, sem):
        s8 = lax.rem(j, 8)
        hoff = lax.rem(j, 3) * _C2
        pltpu.async_copy(hs_v.at[pl.ds(hoff, _C2)],
                         acc_sh.at[dst_r.at[s8]], sem, add=True)

    def scatter_wait(j, sem):
        s8 = lax.rem(j, 8)
        hoff = lax.rem(j, 3) * _C2
        pltpu.make_async_copy(hs_v.at[pl.ds(hoff, _C2)],
                              acc_sh.at[dst_r.at[s8]], sem).wait()

    def start(cs):
        for c in cs:
            c.start()

    def wait(cs):
        for c in cs:
            c.wait()

    start(idx_copies(0, sem_i0))
    start(idx_copies(1, sem_i1))
    wait(idx_copies(0, sem_i0))
    start(idx_copies(2, sem_i0))
    start(gather_copies(0, sem_a))

    def chunk(i, carry):
        par = lax.rem(i, 2)
        hoff = lax.rem(i, 3) * _C2
        roff = par * _C2
        s8 = lax.rem(i, 8)

        @pl.when(jnp.logical_and(par == 0, i + 1 < nchk))
        def _():
            wait(idx_copies(i + 1, sem_i1))

        @pl.when(jnp.logical_and(par == 1, i + 1 < nchk))
        def _():
            wait(idx_copies(i + 1, sem_i0))

        @pl.when(jnp.logical_and(par == 0, i >= 2))
        def _():
            scatter_wait(i - 2, sem_ua)

        @pl.when(jnp.logical_and(par == 1, i >= 2))
        def _():
            scatter_wait(i - 2, sem_ub)

        @pl.when(jnp.logical_and(par == 0, i + 1 < nchk))
        def _():
            start(gather_copies(i + 1, sem_b))

        @pl.when(jnp.logical_and(par == 1, i + 1 < nchk))
        def _():
            start(gather_copies(i + 1, sem_a))

        @pl.when(jnp.logical_and(par == 0, i + 3 < nchk))
        def _():
            start(idx_copies(i + 3, sem_i1))

        @pl.when(jnp.logical_and(par == 1, i + 3 < nchk))
        def _():
            start(idx_copies(i + 3, sem_i0))

        @pl.when(par == 0)
        def _():
            wait(gather_copies(i, sem_a))

        @pl.when(par == 1)
        def _():
            wait(gather_copies(i, sem_b))

        for g in range(_C2 // _L + (1 if _C2 % _L else 0)):
            ne = min(_L, _C2 - g * _L)
            w16 = w_r[s8, pl.ds(g * _L, _L)]
            for e in range(ne):
                row = hoff + g * _L + e
                rrow = roff + g * _L + e
                wb = jnp.full((_L,), w16[e], jnp.float32)
                for k in range(8):
                    s = hs_v[row, pl.ds(k * _L, _L)]
                    r = hr_v[rrow, pl.ds(k * _L, _L)]
                    hs_v[row, pl.ds(k * _L, _L)] = (s + r) * wb

        @pl.when(par == 0)
        def _():
            scatter_start(i, sem_ua)

        @pl.when(par == 1)
        def _():
            scatter_start(i, sem_ub)

        return carry

    lax.fori_loop(0, nchk, chunk, 0)
    scatter_wait(nchk - 2, sem_ua)
    scatter_wait(nchk - 1, sem_ub)
    plsc.subcore_barrier()

    @pl.when(sid < _NS - 1)
    def _():
        pltpu.sync_copy(acc_sh.at[pl.ds(row0, rpt0)],
                        out_hbm.at[cid, pl.ds(row0, rpt0)])

    @pl.when(sid == _NS - 1)
    def _():
        pltpu.sync_copy(acc_sh.at[pl.ds(row0, last)],
                        out_hbm.at[cid, pl.ds(row0, last)])


def _pass_scatter(x, r2, src, typ, dst, w):
    N, H = x.shape
    z = jnp.zeros((N - (N // 8 // _NS) * 8 * (_NS - 1), H), jnp.float32)
    mesh = plsc.VectorSubcoreMesh(
        core_axis_name="c", subcore_axis_name="s",
        num_cores=_NC, num_subcores=_NS)
    f = functools.partial(
        pl.kernel,
        out_type=jax.ShapeDtypeStruct((_NC, N, H), jnp.float32),
        mesh=mesh,
        scratch_types=[
            pltpu.VMEM((8, _C2), jnp.int32),
            pltpu.VMEM((8, _C2), jnp.int32),
            pltpu.VMEM((8, _C2), jnp.int32),
            pltpu.VMEM((8, _C2 + _L), jnp.float32),
            pltpu.VMEM((3 * _C2, H), jnp.float32),
            pltpu.VMEM((2 * _C2, H), jnp.float32),
            pltpu.VMEM_SHARED((N, H), jnp.float32),
            pltpu.SemaphoreType.DMA,
            pltpu.SemaphoreType.DMA,
            pltpu.SemaphoreType.DMA,
            pltpu.SemaphoreType.DMA,
            pltpu.SemaphoreType.DMA,
            pltpu.SemaphoreType.DMA,
        ],
    )(_scatter_body)
    return f(x, r2, src, typ, dst, w, z)


def kernel(x, edge_index, edge_type, rel_emb, res_att, W_ww, W_rel):
    E = edge_type.shape[0]
    nchk = E // _NW // _C
    r2, o2 = _matmuls(rel_emb, W_ww, W_rel)
    src = edge_index[0]
    dst = edge_index[1]
    src3 = src.reshape(_NW, nchk, _C)
    dst3 = dst.reshape(_NW, nchk, _C)
    typ3 = edge_type.reshape(_NW, nchk, _C)
    dpp = _pass_dp(x, r2, src3, typ3, dst3)
    w = _softmax_w(dpp.reshape(E, _L))
    partial = _pass_scatter(x, r2, src, edge_type, dst, w)
    x_e = _combine(partial)
    return (x_e, o2, res_att)


# speedup vs baseline: 4.7770x; 1.3703x over previous
import functools

import jax
import jax.numpy as jnp
from jax import lax
from jax.experimental import pallas as pl
from jax.experimental.pallas import tpu as pltpu
from jax.experimental.pallas import tpu_sc as plsc

_NC = 2
_NS = 16
_NW = _NC * _NS
_L = 16
_C = 80
_C2 = 40
_KS = 25


def _tile_rows(n, sid):
    rpt0 = (n // 8 // _NS) * 8
    last = n - rpt0 * (_NS - 1)
    return rpt0, last, sid * rpt0


def _stage_rows(src_hbm, dst_sh, sid):
    n = src_hbm.shape[0]
    rpt0, last, row0 = _tile_rows(n, sid)

    @pl.when(sid < _NS - 1)
    def _():
        pltpu.sync_copy(src_hbm.at[pl.ds(row0, rpt0)],
                        dst_sh.at[pl.ds(row0, rpt0)])

    @pl.when(sid == _NS - 1)
    def _():
        pltpu.sync_copy(src_hbm.at[pl.ds(row0, last)],
                        dst_sh.at[pl.ds(row0, last)])


def _mm_body(rel_ref, ww_ref, wrel_ref, r2_ref, o2_ref):
    r = rel_ref[...]
    r2_ref[...] = lax.dot(r, ww_ref[...], precision=lax.Precision.HIGHEST)
    o2_ref[...] = lax.dot(r, wrel_ref[...], precision=lax.Precision.HIGHEST)


def _matmuls(rel_emb, W_ww, W_rel):
    R, H = rel_emb.shape
    return pl.pallas_call(
        _mm_body,
        out_shape=[
            jax.ShapeDtypeStruct((R, H), jnp.float32),
            jax.ShapeDtypeStruct((R, H), jnp.float32),
        ],
    )(rel_emb, W_ww, W_rel)


def _smax_stat_body(dpp_ref, m_ref, s_ref, msc, ssc):
    i = pl.program_id(0)
    d = jnp.sum(dpp_ref[0], axis=1)
    bm = jnp.max(d)
    bs = jnp.sum(jnp.exp(d - bm))

    @pl.when(i == 0)
    def _():
        msc[0] = bm
        ssc[0] = bs

    @pl.when(i > 0)
    def _():
        om = msc[0]
        os = ssc[0]
        nm = jnp.maximum(om, bm)
        ssc[0] = os * jnp.exp(om - nm) + bs * jnp.exp(bm - nm)
        msc[0] = nm

    @pl.when(i == pl.num_programs(0) - 1)
    def _():
        m_ref[0] = msc[0]
        s_ref[0] = ssc[0]


def _smax_norm_body(dpp_ref, m_ref, s_ref, w_ref):
    d = jnp.sum(dpp_ref[0], axis=1)
    w_ref[...] = (jnp.exp(d - m_ref[0]) / s_ref[0]).reshape(w_ref.shape)


def _softmax_w(dpp):
    E = dpp.shape[0]
    B = E // _KS
    dpp3 = dpp.reshape(_KS, B, _L)
    m, s = pl.pallas_call(
        _smax_stat_body,
        grid=(_KS,),
        in_specs=[pl.BlockSpec((1, B, _L), lambda i: (i, 0, 0))],
        out_specs=[
            pl.BlockSpec(memory_space=pltpu.SMEM),
            pl.BlockSpec(memory_space=pltpu.SMEM),
        ],
        out_shape=[
            jax.ShapeDtypeStruct((1,), jnp.float32),
            jax.ShapeDtypeStruct((1,), jnp.float32),
        ],
        scratch_shapes=[
            pltpu.SMEM((1,), jnp.float32),
            pltpu.SMEM((1,), jnp.float32),
        ],
    )(dpp3)
    w = pl.pallas_call(
        _smax_norm_body,
        grid=(_KS,),
        in_specs=[
            pl.BlockSpec((1, B, _L), lambda i: (i, 0, 0)),
            pl.BlockSpec(memory_space=pltpu.SMEM),
            pl.BlockSpec(memory_space=pltpu.SMEM),
        ],
        out_specs=pl.BlockSpec((1, B // 128, 128), lambda i: (i, 0, 0)),
        out_shape=jax.ShapeDtypeStruct((_KS, B // 128, 128), jnp.float32),
    )(dpp3, m, s)
    return w.reshape(E)


def _combine_body(p_ref, o_ref):
    o_ref[...] = jnp.maximum(p_ref[0] + p_ref[1], 0.0)


def _combine(partial):
    _, N, H = partial.shape
    return pl.pallas_call(
        _combine_body,
        out_shape=jax.ShapeDtypeStruct((N, H), jnp.float32),
    )(partial)


def _dp_body(x_hbm, r2_hbm, src_hbm, typ_hbm, dst_hbm, dpp_hbm,
             src_a, typ_a, dst_a,
             hs_v, hr_v, ht_v, pacc_v,
             sem_a, sem_b):
    nchk = src_hbm.shape[1]
    cid = lax.axis_index("c")
    sid = lax.axis_index("s")
    wid = sid * _NC + cid

    pltpu.sync_copy(src_hbm.at[wid], src_a)
    pltpu.sync_copy(typ_hbm.at[wid], typ_a)
    pltpu.sync_copy(dst_hbm.at[wid], dst_a)
    plsc.subcore_barrier()

    def start_gathers(i, off, sem):
        pltpu.async_copy(x_hbm.at[src_a.at[i]], hs_v.at[pl.ds(off, _C)], sem)
        pltpu.async_copy(r2_hbm.at[typ_a.at[i]], hr_v.at[pl.ds(off, _C)], sem)
        pltpu.async_copy(x_hbm.at[dst_a.at[i]], ht_v.at[pl.ds(off, _C)], sem)

    def wait_gathers(i, off, sem):
        pltpu.make_async_copy(
            x_hbm.at[src_a.at[i]], hs_v.at[pl.ds(off, _C)], sem).wait()
        pltpu.make_async_copy(
            r2_hbm.at[typ_a.at[i]], hr_v.at[pl.ds(off, _C)], sem).wait()
        pltpu.make_async_copy(
            x_hbm.at[dst_a.at[i]], ht_v.at[pl.ds(off, _C)], sem).wait()

    start_gathers(0, 0, sem_a)

    def chunk(i, carry):
        par = lax.rem(i, 2)
        off = par * _C

        @pl.when(par == 0)
        def _():
            wait_gathers(i, 0, sem_a)

        @pl.when(par == 1)
        def _():
            wait_gathers(i, _C, sem_b)

        @pl.when(jnp.logical_and(par == 0, i + 1 < nchk))
        def _():
            start_gathers(i + 1, _C, sem_b)

        @pl.when(jnp.logical_and(par == 1, i + 1 < nchk))
        def _():
            start_gathers(i + 1, 0, sem_a)

        def group(g, c2):
            for e in range(_L):
                row = off + g * _L + e
                acc = jnp.zeros((_L,), jnp.float32)
                for k in range(8):
                    s = hs_v[row, pl.ds(k * _L, _L)]
                    r = hr_v[row, pl.ds(k * _L, _L)]
                    t = ht_v[row, pl.ds(k * _L, _L)]
                    acc = acc + (s + r) * t
                pacc_v[row, :] = acc
            return c2

        lax.fori_loop(0, _C // _L, group, 0)
        pltpu.sync_copy(pacc_v.at[pl.ds(off, _C)], dpp_hbm.at[wid, i])
        return carry

    lax.fori_loop(0, nchk, chunk, 0)


def _pass_dp(x, r2, src3, typ3, dst3):
    NWk, nchk, C = src3.shape
    N, H = x.shape
    mesh = plsc.VectorSubcoreMesh(
        core_axis_name="c", subcore_axis_name="s",
        num_cores=_NC, num_subcores=_NS)
    f = functools.partial(
        pl.kernel,
        out_type=jax.ShapeDtypeStruct((NWk, nchk, C, _L), jnp.float32),
        mesh=mesh,
        scratch_types=[
            pltpu.VMEM((nchk, C), jnp.int32),
            pltpu.VMEM((nchk, C), jnp.int32),
            pltpu.VMEM((nchk, C), jnp.int32),
            pltpu.VMEM((2 * C, H), jnp.float32),
            pltpu.VMEM((2 * C, H), jnp.float32),
            pltpu.VMEM((2 * C, H), jnp.float32),
            pltpu.VMEM((2 * C, _L), jnp.float32),
            pltpu.SemaphoreType.DMA,
            pltpu.SemaphoreType.DMA,
        ],
    )(_dp_body)
    return f(x, r2, src3, typ3, dst3)


def _scatter_body(x_hbm, r2_hbm, src_hbm, typ_hbm, dst_hbm, w_hbm, z_hbm,
                  out_hbm,
                  src_r, typ_r, dst_r, w_r, hs_v, hr_v, upd_v,
                  acc_sh, sem_i0, sem_i1, sem_a, sem_b, sem_ua, sem_ub):
    E = src_hbm.shape[0]
    N = acc_sh.shape[0]
    nchk = E // _NW // _C2
    cid = lax.axis_index("c")
    sid = lax.axis_index("s")
    wid = sid * _NC + cid
    base = wid * (E // _NW)
    rpt0, last, row0 = _tile_rows(N, sid)

    @pl.when(sid < _NS - 1)
    def _():
        pltpu.sync_copy(z_hbm.at[pl.ds(0, rpt0)],
                        acc_sh.at[pl.ds(row0, rpt0)])

    @pl.when(sid == _NS - 1)
    def _():
        pltpu.sync_copy(z_hbm, acc_sh.at[pl.ds(row0, last)])

    plsc.subcore_barrier()

    def idx_copies(j, sem):
        s8 = lax.rem(j, 8)
        cb = base + j * _C2
        return [
            pltpu.make_async_copy(src_hbm.at[pl.ds(cb, _C2)],
                                  src_r.at[s8], sem),
            pltpu.make_async_copy(typ_hbm.at[pl.ds(cb, _C2)],
                                  typ_r.at[s8], sem),
            pltpu.make_async_copy(dst_hbm.at[pl.ds(cb, _C2)],
                                  dst_r.at[s8], sem),
            pltpu.make_async_copy(w_hbm.at[pl.ds(cb, _C2)],
                                  w_r.at[s8, pl.ds(0, _C2)], sem),
        ]

    def gather_copies(j, sem):
        s8 = lax.rem(j, 8)
        boff = lax.rem(j, 2) * _C2
        return [
            pltpu.make_async_copy(x_hbm.at[src_r.at[s8]],
                                  hs_v.at[pl.ds(boff, _C2)], sem),
            pltpu.make_async_copy(r2_hbm.at[typ_r.at[s8]],
                                  hr_v.at[pl.ds(boff, _C2)], sem),
        ]

    def scatter_start(j, sem):
        s8 = lax.rem(j, 8)
        boff = lax.rem(j, 2) * _C2
        pltpu.async_copy(upd_v.at[pl.ds(boff, _C2)],
                         acc_sh.at[dst_r.at[s8]], sem, add=True)

    def scatter_wait(j, sem):
        s8 = lax.rem(j, 8)
        boff = lax.rem(j, 2) * _C2
        pltpu.make_async_copy(upd_v.at[pl.ds(boff, _C2)],
                              acc_sh.at[dst_r.at[s8]], sem).wait()

    def start(cs):
        for c in cs:
            c.start()

    def wait(cs):
        for c in cs:
            c.wait()

    start(idx_copies(0, sem_i0))
    start(idx_copies(1, sem_i1))
    wait(idx_copies(0, sem_i0))
    start(idx_copies(2, sem_i0))
    start(gather_copies(0, sem_a))

    def chunk(i, carry):
        par = lax.rem(i, 2)
        boff = par * _C2
        s8 = lax.rem(i, 8)

        @pl.when(jnp.logical_and(par == 0, i + 1 < nchk))
        def _():
            wait(idx_copies(i + 1, sem_i1))

        @pl.when(jnp.logical_and(par == 1, i + 1 < nchk))
        def _():
            wait(idx_copies(i + 1, sem_i0))

        @pl.when(jnp.logical_and(par == 0, i >= 2))
        def _():
            scatter_wait(i - 2, sem_ua)

        @pl.when(jnp.logical_and(par == 1, i >= 2))
        def _():
            scatter_wait(i - 2, sem_ub)

        @pl.when(jnp.logical_and(par == 0, i + 1 < nchk))
        def _():
            start(gather_copies(i + 1, sem_b))

        @pl.when(jnp.logical_and(par == 1, i + 1 < nchk))
        def _():
            start(gather_copies(i + 1, sem_a))

        @pl.when(jnp.logical_and(par == 0, i + 3 < nchk))
        def _():
            start(idx_copies(i + 3, sem_i1))

        @pl.when(jnp.logical_and(par == 1, i + 3 < nchk))
        def _():
            start(idx_copies(i + 3, sem_i0))

        @pl.when(par == 0)
        def _():
            wait(gather_copies(i, sem_a))

        @pl.when(par == 1)
        def _():
            wait(gather_copies(i, sem_b))

        for g in range(_C2 // _L + (1 if _C2 % _L else 0)):
            ne = min(_L, _C2 - g * _L)
            w16 = w_r[s8, pl.ds(g * _L, _L)]
            for e in range(ne):
                row = boff + g * _L + e
                wb = jnp.full((_L,), w16[e], jnp.float32)
                for k in range(8):
                    s = hs_v[row, pl.ds(k * _L, _L)]
                    r = hr_v[row, pl.ds(k * _L, _L)]
                    upd_v[row, pl.ds(k * _L, _L)] = (s + r) * wb

        @pl.when(par == 0)
        def _():
            scatter_start(i, sem_ua)

        @pl.when(par == 1)
        def _():
            scatter_start(i, sem_ub)

        return carry

    lax.fori_loop(0, nchk, chunk, 0)
    scatter_wait(nchk - 2, sem_ua)
    scatter_wait(nchk - 1, sem_ub)
    plsc.subcore_barrier()

    @pl.when(sid < _NS - 1)
    def _():
        pltpu.sync_copy(acc_sh.at[pl.ds(row0, rpt0)],
                        out_hbm.at[cid, pl.ds(row0, rpt0)])

    @pl.when(sid == _NS - 1)
    def _():
        pltpu.sync_copy(acc_sh.at[pl.ds(row0, last)],
                        out_hbm.at[cid, pl.ds(row0, last)])


def _pass_scatter(x, r2, src, typ, dst, w):
    N, H = x.shape
    z = jnp.zeros((N - (N // 8 // _NS) * 8 * (_NS - 1), H), jnp.float32)
    mesh = plsc.VectorSubcoreMesh(
        core_axis_name="c", subcore_axis_name="s",
        num_cores=_NC, num_subcores=_NS)
    f = functools.partial(
        pl.kernel,
        out_type=jax.ShapeDtypeStruct((_NC, N, H), jnp.float32),
        mesh=mesh,
        scratch_types=[
            pltpu.VMEM((8, _C2), jnp.int32),
            pltpu.VMEM((8, _C2), jnp.int32),
            pltpu.VMEM((8, _C2), jnp.int32),
            pltpu.VMEM((8, _C2 + _L), jnp.float32),
            pltpu.VMEM((2 * _C2, H), jnp.float32),
            pltpu.VMEM((2 * _C2, H), jnp.float32),
            pltpu.VMEM((2 * _C2, H), jnp.float32),
            pltpu.VMEM_SHARED((N, H), jnp.float32),
            pltpu.SemaphoreType.DMA,
            pltpu.SemaphoreType.DMA,
            pltpu.SemaphoreType.DMA,
            pltpu.SemaphoreType.DMA,
            pltpu.SemaphoreType.DMA,
            pltpu.SemaphoreType.DMA,
        ],
    )(_scatter_body)
    return f(x, r2, src, typ, dst, w, z)


def kernel(x, edge_index, edge_type, rel_emb, res_att, W_ww, W_rel):
    E = edge_type.shape[0]
    nchk = E // _NW // _C
    r2, o2 = _matmuls(rel_emb, W_ww, W_rel)
    src = edge_index[0]
    dst = edge_index[1]
    src3 = src.reshape(_NW, nchk, _C)
    dst3 = dst.reshape(_NW, nchk, _C)
    typ3 = edge_type.reshape(_NW, nchk, _C)
    dpp = _pass_dp(x, r2, src3, typ3, dst3)
    w = _softmax_w(dpp.reshape(E, _L))
    partial = _pass_scatter(x, r2, src, edge_type, dst, w)
    x_e = _combine(partial)
    return (x_e, o2, res_att)
